# baseline (speedup 1.0000x reference)
import functools

import jax
import jax.numpy as jnp
from jax import lax
from jax.experimental import pallas as pl
from jax.experimental.pallas import tpu as pltpu
from jax.experimental.pallas import tpu_sc as plsc

N = 10000
E = 320000
D = 128
PC = 8

NC = 2
NS = 16
NW = NC * NS

CB = 80
NCH = E // CB
CPW = NCH // NW
NSLOT = 3
WIN = 352
TEC_E = 48

ROWS_TC = 1000


def _tables_body(x_ref, posp_ref, wxi_ref, wx_ref, wp_ref, bias_ref,
                 a_ref, b_ref):
    pw = jnp.dot(posp_ref[...], wp_ref[...],
                 preferred_element_type=jnp.float32)
    xw = jnp.dot(x_ref[...], wx_ref[...],
                 preferred_element_type=jnp.float32)
    xi = jnp.dot(x_ref[...], wxi_ref[...],
                 preferred_element_type=jnp.float32)
    a_ref[...] = xw + pw
    b_ref[...] = xi - pw + bias_ref[...]


def _compute_tables(x, posp, w_xi, w_x, w_p, bias):
    return pl.pallas_call(
        _tables_body,
        grid=(N // ROWS_TC,),
        in_specs=[
            pl.BlockSpec((ROWS_TC, D), lambda i: (i, 0)),
            pl.BlockSpec((ROWS_TC, PC), lambda i: (i, 0)),
            pl.BlockSpec((D, D), lambda i: (0, 0)),
            pl.BlockSpec((D, D), lambda i: (0, 0)),
            pl.BlockSpec((PC, D), lambda i: (0, 0)),
            pl.BlockSpec((1, D), lambda i: (0, 0)),
        ],
        out_specs=[
            pl.BlockSpec((ROWS_TC, D), lambda i: (i, 0)),
            pl.BlockSpec((ROWS_TC, D), lambda i: (i, 0)),
        ],
        out_shape=[
            jax.ShapeDtypeStruct((N, D), jnp.float32),
            jax.ShapeDtypeStruct((N, D), jnp.float32),
        ],
    )(x, posp, w_xi, w_x, w_p, bias)


def _edge_body(a_hbm, b_hbm, nbr_hbm, nbbh_hbm, nbbt_hbm, out_hbm, *scr):
    idx_a, idx_bh, idx_bt, bwin = scr[0], scr[1], scr[2], scr[3]
    rows = scr[4:4 + NSLOT]
    sems = scr[4 + NSLOT:4 + 2 * NSLOT]
    sid = lax.axis_index("s")
    wid = sid * NC + lax.axis_index("c")
    ch0 = wid * CPW

    pltpu.sync_copy(nbr_hbm.at[wid], idx_a)
    pltpu.sync_copy(nbbh_hbm.at[wid], idx_bh)
    pltpu.sync_copy(nbbt_hbm.at[wid], idx_bt)

    def start_ga(g, s):
        pltpu.async_copy(a_hbm.at[idx_a.at[g]], rows[s], sems[s])

    def wait_slot(g, s):
        pltpu.make_async_copy(a_hbm.at[idx_a.at[g]], rows[s], sems[s]).wait()

    def start_gb(g, s):
        pltpu.async_copy(b_hbm.at[idx_bh.at[g]],
                         rows[s].at[pl.ds(0, TEC_E)], sems[s], add=True)
        pltpu.async_copy(b_hbm.at[idx_bt.at[g]],
                         rows[s].at[pl.ds(TEC_E, CB - TEC_E)], sems[s],
                         add=True)

    def _out_slice(g):
        off = pl.multiple_of((ch0 + g) * CB, 8)
        return out_hbm.at[pl.ds(off, CB)]

    def start_w(g, s):
        pltpu.async_copy(rows[s], _out_slice(g), sems[s])

    def wait_w(g, s):
        pltpu.make_async_copy(rows[s], _out_slice(g), sems[s]).wait()

    nb_first = idx_bh[0, pl.ds(0, 16)][0]
    nb_last = idx_bt[CPW - 1, pl.ds(CB - TEC_E - 16, 16)][15]
    base = pl.multiple_of(
        jnp.minimum((nb_first // 8) * 8, N - WIN), 8)
    in_window = (nb_last - base) < WIN
    iota16 = lax.iota(jnp.int32, 16)

    @pl.when(in_window)
    def _fast():
        pltpu.sync_copy(b_hbm.at[pl.ds(base, WIN)], bwin)

        def add_b(g, s):
            def ep_body(ep, carry):
                liv = idx_bh[g, pl.ds(ep * 16, 16)] - base
                lis = [liv[k] for k in range(16)]
                e0 = ep * 16
                def col_body(j, carry2):
                    sl = pl.ds(j * 16, 16)
                    for k in range(16):
                        rows[s][e0 + k, sl] = (
                            rows[s][e0 + k, sl] + bwin[lis[k], sl])
                    return carry2
                lax.fori_loop(0, D // 16, col_body, 0)
                return carry
            lax.fori_loop(0, TEC_E // 16, ep_body, 0)

        def start_gbp(g, s):
            pltpu.async_copy(
                b_hbm.at[idx_bt.at[g]],
                rows[s].at[pl.ds(TEC_E, CB - TEC_E)], sems[s], add=True)

        def wait_gbp(g, s):
            pltpu.make_async_copy(
                b_hbm.at[idx_bt.at[g]],
                rows[s].at[pl.ds(TEC_E, CB - TEC_E)], sems[s]).wait()

        def fstep(g, b, steady):
            if steady:
                wait_w(g - NSLOT, b)
            start_ga(g, b)
            p = (b - 1) % NSLOT
            wait_slot(g - 1, p)
            start_gbp(g - 1, p)
            add_b(g - 1, p)
            wait_gbp(g - 1, p)
            start_w(g - 1, p)

        start_ga(0, 0)
        for g in range(1, NSLOT):
            fstep(g, g, steady=False)

        def fgroup(k, carry):
            for b in range(NSLOT):
                fstep(NSLOT * (k + 1) + b, b, steady=True)
            return carry

        lax.fori_loop(0, (CPW - NSLOT) // NSLOT, fgroup, 0)

        for g in range((CPW // NSLOT) * NSLOT, CPW):
            fstep(g, g % NSLOT, steady=True)

        gl = CPW - 1
        sl_ = gl % NSLOT
        wait_slot(gl, sl_)
        start_gbp(gl, sl_)
        add_b(gl, sl_)
        wait_gbp(gl, sl_)
        start_w(gl, sl_)
        for t in range(NSLOT):
            wait_w(CPW - NSLOT + t, (CPW - NSLOT + t) % NSLOT)

    @pl.when(jnp.logical_not(in_window))
    def _slow():
        def stages(g, steady):
            b = g % NSLOT
            if steady:
                wait_w(g - NSLOT, b)
            start_ga(g, b)
            if g < 1:
                return
            wait_slot(g - 1, (g - 1) % NSLOT)
            start_gb(g - 1, (g - 1) % NSLOT)
            if g < 2:
                return
            wait_slot(g - 2, (g - 2) % NSLOT)
            start_w(g - 2, (g - 2) % NSLOT)

        for g in range(NSLOT):
            stages(g, steady=False)

        def step(g, b):
            wait_w(g - NSLOT, b)
            start_ga(g, b)
            wait_slot(g - 1, (b - 1) % NSLOT)
            start_gb(g - 1, (b - 1) % NSLOT)
            wait_slot(g - 2, (b - 2) % NSLOT)
            start_w(g - 2, (b - 2) % NSLOT)

        def group(g0, carry):
            for b in range(NSLOT):
                step(g0 * NSLOT + b, b)
            return carry

        lax.fori_loop(1, CPW // NSLOT, group, 0)

        for g in range((CPW // NSLOT) * NSLOT, CPW):
            step(g, g % NSLOT)

        g = CPW
        wait_slot(g - 1, (g - 1) % NSLOT)
        start_gb(g - 1, (g - 1) % NSLOT)
        wait_slot(g - 2, (g - 2) % NSLOT)
        start_w(g - 2, (g - 2) % NSLOT)
        wait_slot(g - 1, (g - 1) % NSLOT)
        start_w(g - 1, (g - 1) % NSLOT)
        for t in range(NSLOT):
            wait_w(CPW - NSLOT + t, (CPW - NSLOT + t) % NSLOT)


@functools.lru_cache(maxsize=1)
def _edge_kernel():
    return functools.partial(
        pl.kernel,
        mesh=plsc.VectorSubcoreMesh(core_axis_name="c", subcore_axis_name="s",
                                    num_cores=NC, num_subcores=NS),
        out_type=jax.ShapeDtypeStruct((E, D), jnp.float32),
        scratch_types=(
            [pltpu.VMEM((CPW, CB), jnp.int32)]
            + [pltpu.VMEM((CPW, TEC_E), jnp.int32)]
            + [pltpu.VMEM((CPW, CB - TEC_E), jnp.int32)]
            + [pltpu.VMEM((WIN, D), jnp.float32)]
            + [pltpu.VMEM((CB, D), jnp.float32)] * NSLOT
            + [pltpu.SemaphoreType.DMA] * NSLOT
        ),
    )(_edge_body)


def kernel(pos, x, neighbors, neighbor_batch, W_xi, b_xi, W_xn, b_xn):
    w_p = jnp.zeros((PC, D), jnp.float32).at[:3].set(W_xn[:3])
    w_x = W_xn[3:]
    posp = jnp.pad(pos, ((0, 0), (0, PC - 3)))
    bias = (b_xi + b_xn).reshape(1, D)
    a_tab, b_tab = _compute_tables(x, posp, W_xi, w_x, w_p, bias)
    nbr3d = neighbors.reshape(NW, CPW, CB)
    nbb3d = neighbor_batch.reshape(NW, CPW, CB)
    nbbh3d = nbb3d[:, :, :TEC_E]
    nbbt3d = nbb3d[:, :, TEC_E:]
    return _edge_kernel()(a_tab, b_tab, nbr3d, nbbh3d, nbbt3d)


# speedup vs baseline: 1.1376x; 1.1376x over previous
import functools

import jax
import jax.numpy as jnp
from jax import lax
from jax.experimental import pallas as pl
from jax.experimental.pallas import tpu as pltpu
from jax.experimental.pallas import tpu_sc as plsc

N = 10000
E = 320000
D = 128
PC = 8

NC = 2
NS = 16
NW = NC * NS

CB = 80
NCH = E // CB
CPW = NCH // NW
NSLOT = 3
WIN = 352
TEC_E = 64

ROWS_TC = 1000


def _tables_body(x_ref, posp_ref, wxi_ref, wx_ref, wp_ref, bias_ref,
                 a_ref, b_ref):
    pw = jnp.dot(posp_ref[...], wp_ref[...],
                 preferred_element_type=jnp.float32)
    xw = jnp.dot(x_ref[...], wx_ref[...],
                 preferred_element_type=jnp.float32)
    xi = jnp.dot(x_ref[...], wxi_ref[...],
                 preferred_element_type=jnp.float32)
    a_ref[...] = xw + pw
    b_ref[...] = xi - pw + bias_ref[...]


def _compute_tables(x, posp, w_xi, w_x, w_p, bias):
    return pl.pallas_call(
        _tables_body,
        grid=(N // ROWS_TC,),
        in_specs=[
            pl.BlockSpec((ROWS_TC, D), lambda i: (i, 0)),
            pl.BlockSpec((ROWS_TC, PC), lambda i: (i, 0)),
            pl.BlockSpec((D, D), lambda i: (0, 0)),
            pl.BlockSpec((D, D), lambda i: (0, 0)),
            pl.BlockSpec((PC, D), lambda i: (0, 0)),
            pl.BlockSpec((1, D), lambda i: (0, 0)),
        ],
        out_specs=[
            pl.BlockSpec((ROWS_TC, D), lambda i: (i, 0)),
            pl.BlockSpec((ROWS_TC, D), lambda i: (i, 0)),
        ],
        out_shape=[
            jax.ShapeDtypeStruct((N, D), jnp.float32),
            jax.ShapeDtypeStruct((N, D), jnp.float32),
        ],
    )(x, posp, w_xi, w_x, w_p, bias)


def _edge_body(a_hbm, b_hbm, nbr_hbm, nbbh_hbm, nbbt_hbm, out_hbm, *scr):
    idx_a, idx_bh, idx_bt, bwin = scr[0], scr[1], scr[2], scr[3]
    rows = scr[4:4 + NSLOT]
    sems = scr[4 + NSLOT:4 + 2 * NSLOT]
    sid = lax.axis_index("s")
    wid = sid * NC + lax.axis_index("c")
    ch0 = wid * CPW

    pltpu.sync_copy(nbr_hbm.at[wid], idx_a)
    pltpu.sync_copy(nbbh_hbm.at[wid], idx_bh)
    pltpu.sync_copy(nbbt_hbm.at[wid], idx_bt)

    def start_ga(g, s):
        pltpu.async_copy(a_hbm.at[idx_a.at[g]], rows[s], sems[s])

    def wait_slot(g, s):
        pltpu.make_async_copy(a_hbm.at[idx_a.at[g]], rows[s], sems[s]).wait()

    def start_gb(g, s):
        pltpu.async_copy(b_hbm.at[idx_bh.at[g]],
                         rows[s].at[pl.ds(0, TEC_E)], sems[s], add=True)
        pltpu.async_copy(b_hbm.at[idx_bt.at[g]],
                         rows[s].at[pl.ds(TEC_E, CB - TEC_E)], sems[s],
                         add=True)

    def _out_slice(g):
        off = pl.multiple_of((ch0 + g) * CB, 8)
        return out_hbm.at[pl.ds(off, CB)]

    def start_w(g, s):
        pltpu.async_copy(rows[s], _out_slice(g), sems[s])

    def wait_w(g, s):
        pltpu.make_async_copy(rows[s], _out_slice(g), sems[s]).wait()

    nb_first = idx_bh[0, pl.ds(0, 16)][0]
    nb_last = idx_bt[CPW - 1, pl.ds(CB - TEC_E - 16, 16)][15]
    base = pl.multiple_of(
        jnp.minimum((nb_first // 8) * 8, N - WIN), 8)
    in_window = (nb_last - base) < WIN
    iota16 = lax.iota(jnp.int32, 16)

    @pl.when(in_window)
    def _fast():
        pltpu.sync_copy(b_hbm.at[pl.ds(base, WIN)], bwin)

        def add_b(g, s):
            def ep_body(ep, carry):
                liv = idx_bh[g, pl.ds(ep * 16, 16)] - base
                e0 = ep * 16
                l0 = liv[0]
                l15 = liv[15]

                @pl.when(l0 == l15)
                def _uni():
                    def col_u(j, carry2):
                        sl = pl.ds(j * 16, 16)
                        v = bwin[l0, sl]
                        for k in range(16):
                            rows[s][e0 + k, sl] = rows[s][e0 + k, sl] + v
                        return carry2
                    lax.fori_loop(0, D // 16, col_u, 0)

                @pl.when(l0 != l15)
                def _mix():
                    lis = [liv[k] for k in range(16)]
                    def col_m(j, carry2):
                        sl = pl.ds(j * 16, 16)
                        for k in range(16):
                            rows[s][e0 + k, sl] = (
                                rows[s][e0 + k, sl] + bwin[lis[k], sl])
                        return carry2
                    lax.fori_loop(0, D // 16, col_m, 0)
                return carry
            lax.fori_loop(0, TEC_E // 16, ep_body, 0)

        def start_gbp(g, s):
            pltpu.async_copy(
                b_hbm.at[idx_bt.at[g]],
                rows[s].at[pl.ds(TEC_E, CB - TEC_E)], sems[s], add=True)

        def wait_gbp(g, s):
            pltpu.make_async_copy(
                b_hbm.at[idx_bt.at[g]],
                rows[s].at[pl.ds(TEC_E, CB - TEC_E)], sems[s]).wait()

        def fstep(g, b, steady):
            if steady:
                wait_w(g - NSLOT, b)
            start_ga(g, b)
            p = (b - 1) % NSLOT
            wait_slot(g - 1, p)
            start_gbp(g - 1, p)
            add_b(g - 1, p)
            wait_gbp(g - 1, p)
            start_w(g - 1, p)

        start_ga(0, 0)
        for g in range(1, NSLOT):
            fstep(g, g, steady=False)

        def fgroup(k, carry):
            for b in range(NSLOT):
                fstep(NSLOT * (k + 1) + b, b, steady=True)
            return carry

        lax.fori_loop(0, (CPW - NSLOT) // NSLOT, fgroup, 0)

        for g in range((CPW // NSLOT) * NSLOT, CPW):
            fstep(g, g % NSLOT, steady=True)

        gl = CPW - 1
        sl_ = gl % NSLOT
        wait_slot(gl, sl_)
        start_gbp(gl, sl_)
        add_b(gl, sl_)
        wait_gbp(gl, sl_)
        start_w(gl, sl_)
        for t in range(NSLOT):
            wait_w(CPW - NSLOT + t, (CPW - NSLOT + t) % NSLOT)

    @pl.when(jnp.logical_not(in_window))
    def _slow():
        def stages(g, steady):
            b = g % NSLOT
            if steady:
                wait_w(g - NSLOT, b)
            start_ga(g, b)
            if g < 1:
                return
            wait_slot(g - 1, (g - 1) % NSLOT)
            start_gb(g - 1, (g - 1) % NSLOT)
            if g < 2:
                return
            wait_slot(g - 2, (g - 2) % NSLOT)
            start_w(g - 2, (g - 2) % NSLOT)

        for g in range(NSLOT):
            stages(g, steady=False)

        def step(g, b):
            wait_w(g - NSLOT, b)
            start_ga(g, b)
            wait_slot(g - 1, (b - 1) % NSLOT)
            start_gb(g - 1, (b - 1) % NSLOT)
            wait_slot(g - 2, (b - 2) % NSLOT)
            start_w(g - 2, (b - 2) % NSLOT)

        def group(g0, carry):
            for b in range(NSLOT):
                step(g0 * NSLOT + b, b)
            return carry

        lax.fori_loop(1, CPW // NSLOT, group, 0)

        for g in range((CPW // NSLOT) * NSLOT, CPW):
            step(g, g % NSLOT)

        g = CPW
        wait_slot(g - 1, (g - 1) % NSLOT)
        start_gb(g - 1, (g - 1) % NSLOT)
        wait_slot(g - 2, (g - 2) % NSLOT)
        start_w(g - 2, (g - 2) % NSLOT)
        wait_slot(g - 1, (g - 1) % NSLOT)
        start_w(g - 1, (g - 1) % NSLOT)
        for t in range(NSLOT):
            wait_w(CPW - NSLOT + t, (CPW - NSLOT + t) % NSLOT)


@functools.lru_cache(maxsize=1)
def _edge_kernel():
    return functools.partial(
        pl.kernel,
        mesh=plsc.VectorSubcoreMesh(core_axis_name="c", subcore_axis_name="s",
                                    num_cores=NC, num_subcores=NS),
        out_type=jax.ShapeDtypeStruct((E, D), jnp.float32),
        scratch_types=(
            [pltpu.VMEM((CPW, CB), jnp.int32)]
            + [pltpu.VMEM((CPW, TEC_E), jnp.int32)]
            + [pltpu.VMEM((CPW, CB - TEC_E), jnp.int32)]
            + [pltpu.VMEM((WIN, D), jnp.float32)]
            + [pltpu.VMEM((CB, D), jnp.float32)] * NSLOT
            + [pltpu.SemaphoreType.DMA] * NSLOT
        ),
    )(_edge_body)


def kernel(pos, x, neighbors, neighbor_batch, W_xi, b_xi, W_xn, b_xn):
    w_p = jnp.zeros((PC, D), jnp.float32).at[:3].set(W_xn[:3])
    w_x = W_xn[3:]
    posp = jnp.pad(pos, ((0, 0), (0, PC - 3)))
    bias = (b_xi + b_xn).reshape(1, D)
    a_tab, b_tab = _compute_tables(x, posp, W_xi, w_x, w_p, bias)
    nbr3d = neighbors.reshape(NW, CPW, CB)
    nbb3d = neighbor_batch.reshape(NW, CPW, CB)
    nbbh3d = nbb3d[:, :, :TEC_E]
    nbbt3d = nbb3d[:, :, TEC_E:]
    return _edge_kernel()(a_tab, b_tab, nbr3d, nbbh3d, nbbt3d)


# speedup vs baseline: 1.1382x; 1.0005x over previous
import functools

import jax
import jax.numpy as jnp
from jax import lax
from jax.experimental import pallas as pl
from jax.experimental.pallas import tpu as pltpu
from jax.experimental.pallas import tpu_sc as plsc

N = 10000
E = 320000
D = 128
PC = 8

NC = 2
NS = 16
NW = NC * NS

CB = 80
NCH = E // CB
CPW = NCH // NW
NSLOT = 3
WIN = 352
TEC_E = 64

ROWS_TC = 1000


def _tables_body(x_ref, posp_ref, wxi_ref, wx_ref, wp_ref, bias_ref,
                 a_ref, b_ref):
    pw = jnp.dot(posp_ref[...], wp_ref[...],
                 preferred_element_type=jnp.float32)
    xw = jnp.dot(x_ref[...], wx_ref[...],
                 preferred_element_type=jnp.float32)
    xi = jnp.dot(x_ref[...], wxi_ref[...],
                 preferred_element_type=jnp.float32)
    a_ref[...] = xw + pw
    b_ref[...] = xi - pw + bias_ref[...]


def _compute_tables(x, posp, w_xi, w_x, w_p, bias):
    return pl.pallas_call(
        _tables_body,
        grid=(N // ROWS_TC,),
        in_specs=[
            pl.BlockSpec((ROWS_TC, D), lambda i: (i, 0)),
            pl.BlockSpec((ROWS_TC, PC), lambda i: (i, 0)),
            pl.BlockSpec((D, D), lambda i: (0, 0)),
            pl.BlockSpec((D, D), lambda i: (0, 0)),
            pl.BlockSpec((PC, D), lambda i: (0, 0)),
            pl.BlockSpec((1, D), lambda i: (0, 0)),
        ],
        out_specs=[
            pl.BlockSpec((ROWS_TC, D), lambda i: (i, 0)),
            pl.BlockSpec((ROWS_TC, D), lambda i: (i, 0)),
        ],
        out_shape=[
            jax.ShapeDtypeStruct((N, D), jnp.float32),
            jax.ShapeDtypeStruct((N, D), jnp.float32),
        ],
    )(x, posp, w_xi, w_x, w_p, bias)


def _edge_body(a_hbm, b_hbm, nbr_hbm, nbbh_hbm, nbbt_hbm, out_hbm, *scr):
    idx_a, idx_bh, idx_bt, bwin = scr[0], scr[1], scr[2], scr[3]
    rows = scr[4:4 + NSLOT]
    sems = scr[4 + NSLOT:4 + 2 * NSLOT]
    sid = lax.axis_index("s")
    wid = sid * NC + lax.axis_index("c")
    ch0 = wid * CPW

    pltpu.sync_copy(nbr_hbm.at[wid], idx_a)
    pltpu.sync_copy(nbbh_hbm.at[wid], idx_bh)
    pltpu.sync_copy(nbbt_hbm.at[wid], idx_bt)

    def start_ga(g, s):
        pltpu.async_copy(a_hbm.at[idx_a.at[g]], rows[s], sems[s])

    def wait_slot(g, s):
        pltpu.make_async_copy(a_hbm.at[idx_a.at[g]], rows[s], sems[s]).wait()

    def start_gb(g, s):
        pltpu.async_copy(b_hbm.at[idx_bh.at[g]],
                         rows[s].at[pl.ds(0, TEC_E)], sems[s], add=True)
        pltpu.async_copy(b_hbm.at[idx_bt.at[g]],
                         rows[s].at[pl.ds(TEC_E, CB - TEC_E)], sems[s],
                         add=True)

    def _out_slice(g):
        off = pl.multiple_of((ch0 + g) * CB, 8)
        return out_hbm.at[pl.ds(off, CB)]

    def start_w(g, s):
        pltpu.async_copy(rows[s], _out_slice(g), sems[s])

    def wait_w(g, s):
        pltpu.make_async_copy(rows[s], _out_slice(g), sems[s]).wait()

    nb_first = idx_bh[0, pl.ds(0, 16)][0]
    nb_last = idx_bt[CPW - 1, pl.ds(CB - TEC_E - 16, 16)][15]
    base = pl.multiple_of(
        jnp.minimum((nb_first // 8) * 8, N - WIN), 8)
    in_window = (nb_last - base) < WIN

    @pl.when(in_window)
    def _fast():
        pltpu.sync_copy(b_hbm.at[pl.ds(base, WIN)], bwin)

        def add_b(g, s):
            def ep_body(ep, carry):
                liv = idx_bh[g, pl.ds(ep * 16, 16)] - base
                e0 = ep * 16
                l0 = liv[0]
                l15 = liv[15]

                @pl.when(l0 == l15)
                def _uni():
                    def col_u(j, carry2):
                        sl = pl.ds(j * 16, 16)
                        v = bwin[l0, sl]
                        for k in range(16):
                            rows[s][e0 + k, sl] = rows[s][e0 + k, sl] + v
                        return carry2
                    lax.fori_loop(0, D // 16, col_u, 0)

                @pl.when(l0 != l15)
                def _mix():
                    lis = [liv[k] for k in range(16)]
                    def col_m(j, carry2):
                        sl = pl.ds(j * 16, 16)
                        for k in range(16):
                            rows[s][e0 + k, sl] = (
                                rows[s][e0 + k, sl] + bwin[lis[k], sl])
                        return carry2
                    lax.fori_loop(0, D // 16, col_m, 0)
                return carry
            lax.fori_loop(0, TEC_E // 16, ep_body, 0)

        def start_gbp(g, s):
            pltpu.async_copy(
                b_hbm.at[idx_bt.at[g]],
                rows[s].at[pl.ds(TEC_E, CB - TEC_E)], sems[s], add=True)

        def wait_gbp(g, s):
            pltpu.make_async_copy(
                b_hbm.at[idx_bt.at[g]],
                rows[s].at[pl.ds(TEC_E, CB - TEC_E)], sems[s]).wait()

        def fstep(g, b, steady):
            if steady:
                wait_w(g - NSLOT, b)
            start_ga(g, b)
            p = (b - 1) % NSLOT
            wait_slot(g - 1, p)
            start_gbp(g - 1, p)
            add_b(g - 1, p)
            wait_gbp(g - 1, p)
            start_w(g - 1, p)

        start_ga(0, 0)
        for g in range(1, NSLOT):
            fstep(g, g, steady=False)

        def fgroup(k, carry):
            for b in range(NSLOT):
                fstep(NSLOT * (k + 1) + b, b, steady=True)
            return carry

        lax.fori_loop(0, (CPW - NSLOT) // NSLOT, fgroup, 0)

        for g in range((CPW // NSLOT) * NSLOT, CPW):
            fstep(g, g % NSLOT, steady=True)

        gl = CPW - 1
        sl_ = gl % NSLOT
        wait_slot(gl, sl_)
        start_gbp(gl, sl_)
        add_b(gl, sl_)
        wait_gbp(gl, sl_)
        start_w(gl, sl_)
        for t in range(NSLOT):
            wait_w(CPW - NSLOT + t, (CPW - NSLOT + t) % NSLOT)

    @pl.when(jnp.logical_not(in_window))
    def _slow():
        def stages(g, steady):
            b = g % NSLOT
            if steady:
                wait_w(g - NSLOT, b)
            start_ga(g, b)
            if g < 1:
                return
            wait_slot(g - 1, (g - 1) % NSLOT)
            start_gb(g - 1, (g - 1) % NSLOT)
            if g < 2:
                return
            wait_slot(g - 2, (g - 2) % NSLOT)
            start_w(g - 2, (g - 2) % NSLOT)

        for g in range(NSLOT):
            stages(g, steady=False)

        def step(g, b):
            wait_w(g - NSLOT, b)
            start_ga(g, b)
            wait_slot(g - 1, (b - 1) % NSLOT)
            start_gb(g - 1, (b - 1) % NSLOT)
            wait_slot(g - 2, (b - 2) % NSLOT)
            start_w(g - 2, (b - 2) % NSLOT)

        def group(g0, carry):
            for b in range(NSLOT):
                step(g0 * NSLOT + b, b)
            return carry

        lax.fori_loop(1, CPW // NSLOT, group, 0)

        for g in range((CPW // NSLOT) * NSLOT, CPW):
            step(g, g % NSLOT)

        g = CPW
        wait_slot(g - 1, (g - 1) % NSLOT)
        start_gb(g - 1, (g - 1) % NSLOT)
        wait_slot(g - 2, (g - 2) % NSLOT)
        start_w(g - 2, (g - 2) % NSLOT)
        wait_slot(g - 1, (g - 1) % NSLOT)
        start_w(g - 1, (g - 1) % NSLOT)
        for t in range(NSLOT):
            wait_w(CPW - NSLOT + t, (CPW - NSLOT + t) % NSLOT)


@functools.lru_cache(maxsize=1)
def _edge_kernel():
    return functools.partial(
        pl.kernel,
        mesh=plsc.VectorSubcoreMesh(core_axis_name="c", subcore_axis_name="s",
                                    num_cores=NC, num_subcores=NS),
        out_type=jax.ShapeDtypeStruct((E, D), jnp.float32),
        scratch_types=(
            [pltpu.VMEM((CPW, CB), jnp.int32)]
            + [pltpu.VMEM((CPW, TEC_E), jnp.int32)]
            + [pltpu.VMEM((CPW, CB - TEC_E), jnp.int32)]
            + [pltpu.VMEM((WIN, D), jnp.float32)]
            + [pltpu.VMEM((CB, D), jnp.float32)] * NSLOT
            + [pltpu.SemaphoreType.DMA] * NSLOT
        ),
    )(_edge_body)


def kernel(pos, x, neighbors, neighbor_batch, W_xi, b_xi, W_xn, b_xn):
    w_p = jnp.zeros((PC, D), jnp.float32).at[:3].set(W_xn[:3])
    w_x = W_xn[3:]
    posp = jnp.pad(pos, ((0, 0), (0, PC - 3)))
    bias = (b_xi + b_xn).reshape(1, D)
    a_tab, b_tab = _compute_tables(x, posp, W_xi, w_x, w_p, bias)
    nbr3d = neighbors.reshape(NW, CPW, CB)
    nbb3d = neighbor_batch.reshape(NW, CPW, CB)
    nbbh3d = nbb3d[:, :, :TEC_E]
    nbbt3d = nbb3d[:, :, TEC_E:]
    return _edge_kernel()(a_tab, b_tab, nbr3d, nbbh3d, nbbt3d)
